# baseline (speedup 1.0000x reference)
import functools

import jax
import jax.numpy as jnp
from jax import lax
from jax.experimental import pallas as pl
from jax.experimental.pallas import tpu as pltpu
from jax.experimental.pallas import tpu_sc as plsc

N_IN = 10000
UP = 8
N_UP = N_IN * UP
E_UP = 1280000
C_IN = 64
C_HID = 64
C_OUT = 32
HALF = 16
L_BLOCK = 3

G_HID = UP * C_HID
G_OUT = UP * C_OUT
G_HALF = UP * HALF

NS = 16
K_CH = 400
NSLOT = 6

_R = 2000
_NG = N_IN // _R


def _up_body(x_ref, w_ref, b_ref, o_ref):
    o_ref[...] = jnp.maximum(
        jnp.dot(x_ref[...], w_ref[...], preferred_element_type=jnp.float32)
        + b_ref[...], 0.0)


def _up(x, wf, bf):
    R = 1000
    return pl.pallas_call(
        _up_body,
        grid=(N_IN // R,),
        in_specs=[pl.BlockSpec((R, C_IN), lambda i: (i, 0)),
                  pl.BlockSpec((C_IN, G_HID), lambda i: (0, 0)),
                  pl.BlockSpec((1, G_HID), lambda i: (0, 0))],
        out_specs=pl.BlockSpec((R, G_HID), lambda i: (i, 0)),
        out_shape=jax.ShapeDtypeStruct((N_IN, G_HID), jnp.float32),
    )(x, wf, bf)


def _proj0_body(h_ref, ws_ref, wlo_ref, whi_ref, b_ref,
                s_ref, tlo_ref, thi_ref):
    hh = h_ref[...]
    s_ref[...] = (jnp.dot(hh, ws_ref[...], preferred_element_type=jnp.float32)
                  + b_ref[...])
    tlo_ref[...] = jnp.dot(hh, wlo_ref[...], preferred_element_type=jnp.float32)
    thi_ref[...] = jnp.dot(hh, whi_ref[...], preferred_element_type=jnp.float32)


def _proj0(h, ws, wlo, whi, b):
    C = h.shape[1]
    return pl.pallas_call(
        _proj0_body,
        grid=(_NG,),
        in_specs=[pl.BlockSpec((_R, C), lambda i: (i, 0)),
                  pl.BlockSpec((C, G_OUT), lambda i: (0, 0)),
                  pl.BlockSpec((C, G_HALF), lambda i: (0, 0)),
                  pl.BlockSpec((C, G_HALF), lambda i: (0, 0)),
                  pl.BlockSpec((1, G_OUT), lambda i: (0, 0))],
        out_specs=[pl.BlockSpec((_R, G_OUT), lambda i: (i, 0)),
                   pl.BlockSpec((_R, G_HALF), lambda i: (i, 0)),
                   pl.BlockSpec((_R, G_HALF), lambda i: (i, 0))],
        out_shape=[jax.ShapeDtypeStruct((N_IN, G_OUT), jnp.float32),
                   jax.ShapeDtypeStruct((N_IN, G_HALF), jnp.float32),
                   jax.ShapeDtypeStruct((N_IN, G_HALF), jnp.float32)],
    )(h, ws, wlo, whi, b)


def _cproj_body(s_ref, ylo_ref, yhi_ref, plo_ref, phi_ref,
                ws_ref, wlo_ref, whi_ref, b_ref,
                h_ref, s2_ref, tlo_ref, thi_ref):
    y = (jnp.dot(ylo_ref[...], plo_ref[...], preferred_element_type=jnp.float32)
         + jnp.dot(yhi_ref[...], phi_ref[...], preferred_element_type=jnp.float32))
    h = jnp.maximum(s_ref[...] + y, 0.0)
    h_ref[...] = h
    s2_ref[...] = (jnp.dot(h, ws_ref[...], preferred_element_type=jnp.float32)
                   + b_ref[...])
    tlo_ref[...] = jnp.dot(h, wlo_ref[...], preferred_element_type=jnp.float32)
    thi_ref[...] = jnp.dot(h, whi_ref[...], preferred_element_type=jnp.float32)


def _cproj_res_body(res_ref, s_ref, ylo_ref, yhi_ref, plo_ref, phi_ref,
                    ws_ref, wlo_ref, whi_ref, b_ref,
                    h_ref, s2_ref, tlo_ref, thi_ref):
    y = (jnp.dot(ylo_ref[...], plo_ref[...], preferred_element_type=jnp.float32)
         + jnp.dot(yhi_ref[...], phi_ref[...], preferred_element_type=jnp.float32))
    h = jnp.maximum(res_ref[...] + s_ref[...] + y, 0.0)
    h_ref[...] = h
    s2_ref[...] = (jnp.dot(h, ws_ref[...], preferred_element_type=jnp.float32)
                   + b_ref[...])
    tlo_ref[...] = jnp.dot(h, wlo_ref[...], preferred_element_type=jnp.float32)
    thi_ref[...] = jnp.dot(h, whi_ref[...], preferred_element_type=jnp.float32)


def _row_spec(w):
    return pl.BlockSpec((_R, w), lambda i: (i, 0))


def _full_spec(r, w):
    return pl.BlockSpec((r, w), lambda i: (0, 0))


_CPROJ_OUT = [jax.ShapeDtypeStruct((N_IN, G_OUT), jnp.float32),
              jax.ShapeDtypeStruct((N_IN, G_OUT), jnp.float32),
              jax.ShapeDtypeStruct((N_IN, G_HALF), jnp.float32),
              jax.ShapeDtypeStruct((N_IN, G_HALF), jnp.float32)]

_CPROJ_OUT_SPECS = [pl.BlockSpec((_R, G_OUT), lambda i: (i, 0)),
                    pl.BlockSpec((_R, G_OUT), lambda i: (i, 0)),
                    pl.BlockSpec((_R, G_HALF), lambda i: (i, 0)),
                    pl.BlockSpec((_R, G_HALF), lambda i: (i, 0))]


def _cproj(s, ylo, yhi, plo, phi, ws, wlo, whi, b):
    return pl.pallas_call(
        _cproj_body,
        grid=(_NG,),
        in_specs=[_row_spec(G_OUT), _row_spec(G_HALF), _row_spec(G_HALF),
                  _full_spec(G_HALF, G_OUT), _full_spec(G_HALF, G_OUT),
                  _full_spec(G_OUT, G_OUT), _full_spec(G_OUT, G_HALF),
                  _full_spec(G_OUT, G_HALF), _full_spec(1, G_OUT)],
        out_specs=_CPROJ_OUT_SPECS,
        out_shape=_CPROJ_OUT,
    )(s, ylo, yhi, plo, phi, ws, wlo, whi, b)


def _cproj_res(res, s, ylo, yhi, plo, phi, ws, wlo, whi, b):
    return pl.pallas_call(
        _cproj_res_body,
        grid=(_NG,),
        in_specs=[_row_spec(G_OUT),
                  _row_spec(G_OUT), _row_spec(G_HALF), _row_spec(G_HALF),
                  _full_spec(G_HALF, G_OUT), _full_spec(G_HALF, G_OUT),
                  _full_spec(G_OUT, G_OUT), _full_spec(G_OUT, G_HALF),
                  _full_spec(G_OUT, G_HALF), _full_spec(1, G_OUT)],
        out_specs=_CPROJ_OUT_SPECS,
        out_shape=_CPROJ_OUT,
    )(res, s, ylo, yhi, plo, phi, ws, wlo, whi, b)


def _ccls_body(res_ref, s_ref, ylo_ref, yhi_ref, plo_ref, phi_ref,
               wcs_ref, wcn_ref, bc_ref,
               h_ref, sc_ref, tpad_ref):
    y = (jnp.dot(ylo_ref[...], plo_ref[...], preferred_element_type=jnp.float32)
         + jnp.dot(yhi_ref[...], phi_ref[...], preferred_element_type=jnp.float32))
    h = jnp.maximum(res_ref[...] + s_ref[...] + y, 0.0)
    h_ref[...] = h
    sc_ref[...] = (jnp.dot(h, wcs_ref[...], preferred_element_type=jnp.float32)
                   + bc_ref[...])
    tpad_ref[...] = jnp.dot(h, wcn_ref[...], preferred_element_type=jnp.float32)


def _ccls(res, s, ylo, yhi, plo, phi, wcs, wcn, bc):
    return pl.pallas_call(
        _ccls_body,
        grid=(_NG,),
        in_specs=[_row_spec(G_OUT),
                  _row_spec(G_OUT), _row_spec(G_HALF), _row_spec(G_HALF),
                  _full_spec(G_HALF, G_OUT), _full_spec(G_HALF, G_OUT),
                  _full_spec(G_OUT, G_HALF), _full_spec(G_OUT, G_HALF),
                  _full_spec(1, G_HALF)],
        out_specs=[pl.BlockSpec((_R, G_OUT), lambda i: (i, 0)),
                   pl.BlockSpec((_R, G_HALF), lambda i: (i, 0)),
                   pl.BlockSpec((_R, G_HALF), lambda i: (i, 0))],
        out_shape=[jax.ShapeDtypeStruct((N_IN, G_OUT), jnp.float32),
                   jax.ShapeDtypeStruct((N_IN, G_HALF), jnp.float32),
                   jax.ShapeDtypeStruct((N_IN, G_HALF), jnp.float32)],
    )(res, s, ylo, yhi, plo, phi, wcs, wcn, bc)


def _finreduce_body(sc_ref, y0_ref, y1_ref, lg_ref, mx_ref, top_ref):
    lg = sc_ref[...] + y0_ref[...] + y1_ref[...]
    lg_ref[...] = lg
    col = lax.broadcasted_iota(jnp.int32, (N_IN, G_HALF), 1)
    node = (lax.broadcasted_iota(jnp.int32, (N_IN, G_HALF), 0) * UP
            + col // HALF)
    valid = (col % HALF) == 0
    neg = jnp.float32(-3.0e38)
    mx = jnp.max(jnp.where(valid, lg, neg))
    mx_ref[...] = jnp.reshape(mx, (1, 1))
    top_ref[...] = jnp.reshape(
        jnp.min(jnp.where(valid & (lg == mx), node, jnp.int32(2**30))), (1, 1))


def _finreduce(sc, y0, y1):
    return pl.pallas_call(
        _finreduce_body,
        in_specs=[pl.BlockSpec((N_IN, G_HALF), lambda: (0, 0))] * 3,
        out_specs=[pl.BlockSpec((N_IN, G_HALF), lambda: (0, 0)),
                   pl.BlockSpec((1, 1), lambda: (0, 0)),
                   pl.BlockSpec((1, 1), lambda: (0, 0))],
        out_shape=[jax.ShapeDtypeStruct((N_IN, G_HALF), jnp.float32),
                   jax.ShapeDtypeStruct((1, 1), jnp.float32),
                   jax.ShapeDtypeStruct((1, 1), jnp.int32)],
    )(sc, y0, y1)


def _mask_body(h_ref, lg_ref, tgt_ref, mx_ref, top_ref, o_ref, k_ref):
    R = 4000
    pid = pl.program_id(0)
    lg = lg_ref[...]
    iota = lax.broadcasted_iota(jnp.int32, (R, 1), 0) + pid * R
    keep = ((lg > 0.0) | (tgt_ref[...] != 0)
            | ((iota == top_ref[0, 0]) & (mx_ref[0, 0] < 0.0)))
    o_ref[...] = h_ref[...] * keep.astype(jnp.float32)
    k_ref[...] = keep.astype(jnp.int32)


def _mask(h, lgN, tgtN, mx, top):
    R = 4000
    return pl.pallas_call(
        _mask_body,
        grid=(N_UP // R,),
        in_specs=[pl.BlockSpec((R, C_OUT), lambda i: (i, 0)),
                  pl.BlockSpec((R, 1), lambda i: (i, 0)),
                  pl.BlockSpec((R, 1), lambda i: (i, 0)),
                  pl.BlockSpec((1, 1), lambda i: (0, 0)),
                  pl.BlockSpec((1, 1), lambda i: (0, 0))],
        out_specs=[pl.BlockSpec((R, C_OUT), lambda i: (i, 0)),
                   pl.BlockSpec((R, 1), lambda i: (i, 0))],
        out_shape=[jax.ShapeDtypeStruct((N_UP, C_OUT), jnp.float32),
                   jax.ShapeDtypeStruct((N_UP, 1), jnp.int32)],
    )(h, lgN, tgtN, mx, top)


_MESH = plsc.VectorSubcoreMesh(core_axis_name="c", subcore_axis_name="s",
                               num_cores=2, num_subcores=NS)

_SC_SCRATCH = (
    [pltpu.VMEM((K_CH,), jnp.int32) for _ in range(NSLOT)]
    + [pltpu.VMEM((K_CH,), jnp.int32) for _ in range(NSLOT)]
    + [pltpu.VMEM((K_CH, HALF), jnp.float32) for _ in range(NSLOT)]
    + [pltpu.VMEM_SHARED((N_UP, HALF), jnp.float32)]
    + [pltpu.SemaphoreType.DMA] * (3 * NSLOT)
)


def _mk_slots(scr):
    srcvs = scr[0:NSLOT]
    dstvs = scr[NSLOT:2 * NSLOT]
    rows = scr[2 * NSLOT:3 * NSLOT]
    sems = scr[3 * NSLOT + 1:]
    isems, gsems, ssems = (sems[0:NSLOT], sems[NSLOT:2 * NSLOT],
                           sems[2 * NSLOT:3 * NSLOT])
    return [(srcvs[s], dstvs[s], rows[s], isems[s], gsems[s], ssems[s])
            for s in range(NSLOT)]


def _edge_loop(table_h, src_h, dst_h, acc, slots, base0, nch):
    def idx_copies(slot, c):
        srcv, dstv, _, isem, _, _ = slots[slot]
        base = base0 + c * K_CH
        return (pltpu.make_async_copy(src_h.at[pl.ds(base, K_CH)], srcv, isem),
                pltpu.make_async_copy(dst_h.at[pl.ds(base, K_CH)], dstv, isem))

    def stage(c_idx, s_idx, c_gat, s_gat, c_sct, s_sct):
        srcv, dstv, rows, isem, gsem, ssem = slots[s_idx]

        @pl.when((c_idx < nch) & (c_idx >= NSLOT))
        def _():
            pltpu.make_async_copy(rows, acc.at[dstv], ssem).wait()

        @pl.when(c_idx < nch)
        def _():
            for d in idx_copies(s_idx, c_idx):
                d.start()

        srcvg, dstvg, rowsg, isemg, gsemg, ssemg = slots[s_gat]

        @pl.when((c_gat >= 0) & (c_gat < nch))
        def _():
            for d in idx_copies(s_gat, c_gat):
                d.wait()
            pltpu.async_copy(table_h.at[srcvg], rowsg, gsemg)

        srcvs, dstvs, rowss, isems, gsems, ssems = slots[s_sct]

        @pl.when((c_sct >= 0) & (c_sct < nch))
        def _():
            pltpu.make_async_copy(table_h.at[srcvs], rowss, gsems).wait()
            pltpu.async_copy(rowss, acc.at[dstvs], ssems, add=True)

    nsup = (nch + 2 + NSLOT - 1) // NSLOT

    def sup(i, carry):
        g0 = i * NSLOT
        for j in range(NSLOT):
            g = g0 + j
            stage(g, j, g - 1, (j - 1) % NSLOT, g - 2, (j - 2) % NSLOT)
        return carry
    lax.fori_loop(0, nsup, sup, 0)
    for c in range(max(0, nch - NSLOT), nch):
        _, dstvs, rowss, _, _, ssems = slots[c % NSLOT]
        pltpu.make_async_copy(rowss, acc.at[dstvs], ssems).wait()


@functools.partial(
    pl.kernel,
    out_type=[jax.ShapeDtypeStruct((N_UP, HALF), jnp.float32)] * 2,
    mesh=_MESH,
    scratch_types=_SC_SCRATCH,
    compiler_params=pltpu.CompilerParams(use_tc_tiling_on_sc=False),
)
def _sc_wide(tlo_h, thi_h, src_h, dst_h, zer_h, ylo_h, yhi_h, *scr):
    cid = lax.axis_index("c")
    sid = lax.axis_index("s")
    slots = _mk_slots(scr)
    acc = scr[3 * NSLOT]
    rpt = N_UP // NS
    row0 = sid * rpt
    ept = E_UP // NS
    nch = ept // K_CH
    pltpu.sync_copy(zer_h.at[pl.ds(row0, rpt)], acc.at[pl.ds(row0, rpt)])
    plsc.subcore_barrier()

    @pl.when(cid == 0)
    def _():
        _edge_loop(tlo_h, src_h, dst_h, acc, slots, sid * ept, nch)

    @pl.when(cid == 1)
    def _():
        _edge_loop(thi_h, src_h, dst_h, acc, slots, sid * ept, nch)

    plsc.subcore_barrier()

    @pl.when(cid == 0)
    def _():
        pltpu.sync_copy(acc.at[pl.ds(row0, rpt)], ylo_h.at[pl.ds(row0, rpt)])

    @pl.when(cid == 1)
    def _():
        pltpu.sync_copy(acc.at[pl.ds(row0, rpt)], yhi_h.at[pl.ds(row0, rpt)])


@functools.partial(
    pl.kernel,
    out_type=[jax.ShapeDtypeStruct((N_UP, HALF), jnp.float32)] * 2,
    mesh=_MESH,
    scratch_types=_SC_SCRATCH,
    compiler_params=pltpu.CompilerParams(use_tc_tiling_on_sc=False),
)
def _sc_cls(t_h, src_h, dst_h, zer_h, y0_h, y1_h, *scr):
    cid = lax.axis_index("c")
    sid = lax.axis_index("s")
    slots = _mk_slots(scr)
    acc = scr[3 * NSLOT]
    rpt = N_UP // NS
    row0 = sid * rpt
    ept = E_UP // (2 * NS)
    nch = ept // K_CH
    base0 = cid * (E_UP // 2) + sid * ept
    pltpu.sync_copy(zer_h.at[pl.ds(row0, rpt)], acc.at[pl.ds(row0, rpt)])
    plsc.subcore_barrier()
    _edge_loop(t_h, src_h, dst_h, acc, slots, base0, nch)
    plsc.subcore_barrier()

    @pl.when(cid == 0)
    def _():
        pltpu.sync_copy(acc.at[pl.ds(row0, rpt)], y0_h.at[pl.ds(row0, rpt)])

    @pl.when(cid == 1)
    def _():
        pltpu.sync_copy(acc.at[pl.ds(row0, rpt)], y1_h.at[pl.ds(row0, rpt)])


def _bd8(w):
    c, d = w.shape
    return jnp.einsum('jk,cd->jckd', jnp.eye(UP, dtype=w.dtype), w).reshape(
        UP * c, UP * d)


def _grouped(a):
    return a.reshape(N_IN, G_HALF)


def _flat16(a):
    return a.reshape(N_UP, HALF)


def kernel(x, edge_index_up, target_label, W_up, b_up, W1_self, W1_nbr, b1,
           Wb_self, Wb_nbr, bb, Wc_self, Wc_nbr, bc):
    f32 = jnp.float32
    src = edge_index_up[0]
    dst = edge_index_up[1]
    zer = jnp.zeros((N_UP, HALF), f32)

    plo = _bd8(jnp.concatenate(
        [jnp.eye(HALF, dtype=f32), jnp.zeros((HALF, HALF), f32)], axis=1))
    phi = _bd8(jnp.concatenate(
        [jnp.zeros((HALF, HALF), f32), jnp.eye(HALF, dtype=f32)], axis=1))

    wf_up = jnp.transpose(W_up, (1, 0, 2)).reshape(C_IN, G_HID)
    bf_up = jnp.tile(b_up, UP).reshape(1, G_HID)
    h = _up(x, wf_up, bf_up)

    s, tlo, thi = _proj0(h, _bd8(W1_self), _bd8(W1_nbr[:, :HALF]),
                         _bd8(W1_nbr[:, HALF:]),
                         jnp.tile(b1, UP).reshape(1, G_OUT))
    ylo, yhi = _sc_wide(_flat16(tlo), _flat16(thi), src, dst, zer)

    hres, s, tlo, thi = _cproj(
        s, _grouped(ylo), _grouped(yhi), plo, phi,
        _bd8(Wb_self[0, 0]), _bd8(Wb_nbr[0, 0][:, :HALF]),
        _bd8(Wb_nbr[0, 0][:, HALF:]), jnp.tile(bb[0, 0], UP).reshape(1, G_OUT))
    ylo, yhi = _sc_wide(_flat16(tlo), _flat16(thi), src, dst, zer)

    for l in range(L_BLOCK):
        _, s, tlo, thi = _cproj(
            s, _grouped(ylo), _grouped(yhi), plo, phi,
            _bd8(Wb_self[l, 1]), _bd8(Wb_nbr[l, 1][:, :HALF]),
            _bd8(Wb_nbr[l, 1][:, HALF:]),
            jnp.tile(bb[l, 1], UP).reshape(1, G_OUT))
        ylo, yhi = _sc_wide(_flat16(tlo), _flat16(thi), src, dst, zer)
        if l < L_BLOCK - 1:
            hres, s, tlo, thi = _cproj_res(
                hres, s, _grouped(ylo), _grouped(yhi), plo, phi,
                _bd8(Wb_self[l + 1, 0]), _bd8(Wb_nbr[l + 1, 0][:, :HALF]),
                _bd8(Wb_nbr[l + 1, 0][:, HALF:]),
                jnp.tile(bb[l + 1, 0], UP).reshape(1, G_OUT))
            ylo, yhi = _sc_wide(_flat16(tlo), _flat16(thi), src, dst, zer)

    e0 = jnp.zeros((1, HALF), f32).at[0, 0].set(1.0)
    wcs = _bd8(Wc_self @ e0)
    wcn = _bd8(Wc_nbr @ e0)
    bc_g = jnp.tile(bc[0] * e0, (1, UP))
    hfin, sc_g, tpad = _ccls(hres, s, _grouped(ylo), _grouped(yhi), plo, phi,
                             wcs, wcn, bc_g)
    y0, y1 = _sc_cls(_flat16(tpad), src, dst, zer)

    lg_g, mx, top = _finreduce(sc_g, _grouped(y0), _grouped(y1))

    out_cls = lg_g.reshape(N_UP, HALF)[:, :1]
    hN = hfin.reshape(N_UP, C_OUT)
    tgtN = target_label.astype(jnp.int32).reshape(N_UP, 1)
    out_pruned, keep_i = _mask(hN, out_cls, tgtN, mx, top)
    keep = keep_i.reshape(N_UP) != 0
    return out_pruned, out_cls, target_label, keep


# speedup vs baseline: 1.0642x; 1.0642x over previous
import functools

import jax
import jax.numpy as jnp
from jax import lax
from jax.experimental import pallas as pl
from jax.experimental.pallas import tpu as pltpu
from jax.experimental.pallas import tpu_sc as plsc

N_IN = 10000
UP = 8
N_UP = N_IN * UP
E_UP = 1280000
C_IN = 64
C_HID = 64
C_OUT = 32
HALF = 16
L_BLOCK = 3

G_HID = UP * C_HID
G_OUT = UP * C_OUT
G_HALF = UP * HALF

NS = 16
K_CH = 800
NSLOT = 3

_R = 2000
_NG = N_IN // _R


def _up_body(x_ref, w_ref, b_ref, o_ref):
    o_ref[...] = jnp.maximum(
        jnp.dot(x_ref[...], w_ref[...], preferred_element_type=jnp.float32)
        + b_ref[...], 0.0)


def _up(x, wf, bf):
    R = 1000
    return pl.pallas_call(
        _up_body,
        grid=(N_IN // R,),
        in_specs=[pl.BlockSpec((R, C_IN), lambda i: (i, 0)),
                  pl.BlockSpec((C_IN, G_HID), lambda i: (0, 0)),
                  pl.BlockSpec((1, G_HID), lambda i: (0, 0))],
        out_specs=pl.BlockSpec((R, G_HID), lambda i: (i, 0)),
        out_shape=jax.ShapeDtypeStruct((N_IN, G_HID), jnp.float32),
    )(x, wf, bf)


def _proj0_body(h_ref, ws_ref, wlo_ref, whi_ref, b_ref,
                s_ref, tlo_ref, thi_ref):
    hh = h_ref[...]
    s_ref[...] = (jnp.dot(hh, ws_ref[...], preferred_element_type=jnp.float32)
                  + b_ref[...])
    tlo_ref[...] = jnp.dot(hh, wlo_ref[...], preferred_element_type=jnp.float32)
    thi_ref[...] = jnp.dot(hh, whi_ref[...], preferred_element_type=jnp.float32)


def _proj0(h, ws, wlo, whi, b):
    C = h.shape[1]
    return pl.pallas_call(
        _proj0_body,
        grid=(_NG,),
        in_specs=[pl.BlockSpec((_R, C), lambda i: (i, 0)),
                  pl.BlockSpec((C, G_OUT), lambda i: (0, 0)),
                  pl.BlockSpec((C, G_HALF), lambda i: (0, 0)),
                  pl.BlockSpec((C, G_HALF), lambda i: (0, 0)),
                  pl.BlockSpec((1, G_OUT), lambda i: (0, 0))],
        out_specs=[pl.BlockSpec((_R, G_OUT), lambda i: (i, 0)),
                   pl.BlockSpec((_R, G_HALF), lambda i: (i, 0)),
                   pl.BlockSpec((_R, G_HALF), lambda i: (i, 0))],
        out_shape=[jax.ShapeDtypeStruct((N_IN, G_OUT), jnp.float32),
                   jax.ShapeDtypeStruct((N_IN, G_HALF), jnp.float32),
                   jax.ShapeDtypeStruct((N_IN, G_HALF), jnp.float32)],
    )(h, ws, wlo, whi, b)


def _cproj_body(s_ref, ylo_ref, yhi_ref, plo_ref, phi_ref,
                ws_ref, wlo_ref, whi_ref, b_ref,
                h_ref, s2_ref, tlo_ref, thi_ref):
    y = (jnp.dot(ylo_ref[...], plo_ref[...], preferred_element_type=jnp.float32)
         + jnp.dot(yhi_ref[...], phi_ref[...], preferred_element_type=jnp.float32))
    h = jnp.maximum(s_ref[...] + y, 0.0)
    h_ref[...] = h
    s2_ref[...] = (jnp.dot(h, ws_ref[...], preferred_element_type=jnp.float32)
                   + b_ref[...])
    tlo_ref[...] = jnp.dot(h, wlo_ref[...], preferred_element_type=jnp.float32)
    thi_ref[...] = jnp.dot(h, whi_ref[...], preferred_element_type=jnp.float32)


def _cproj_res_body(res_ref, s_ref, ylo_ref, yhi_ref, plo_ref, phi_ref,
                    ws_ref, wlo_ref, whi_ref, b_ref,
                    h_ref, s2_ref, tlo_ref, thi_ref):
    y = (jnp.dot(ylo_ref[...], plo_ref[...], preferred_element_type=jnp.float32)
         + jnp.dot(yhi_ref[...], phi_ref[...], preferred_element_type=jnp.float32))
    h = jnp.maximum(res_ref[...] + s_ref[...] + y, 0.0)
    h_ref[...] = h
    s2_ref[...] = (jnp.dot(h, ws_ref[...], preferred_element_type=jnp.float32)
                   + b_ref[...])
    tlo_ref[...] = jnp.dot(h, wlo_ref[...], preferred_element_type=jnp.float32)
    thi_ref[...] = jnp.dot(h, whi_ref[...], preferred_element_type=jnp.float32)


def _row_spec(w):
    return pl.BlockSpec((_R, w), lambda i: (i, 0))


def _full_spec(r, w):
    return pl.BlockSpec((r, w), lambda i: (0, 0))


_CPROJ_OUT = [jax.ShapeDtypeStruct((N_IN, G_OUT), jnp.float32),
              jax.ShapeDtypeStruct((N_IN, G_OUT), jnp.float32),
              jax.ShapeDtypeStruct((N_IN, G_HALF), jnp.float32),
              jax.ShapeDtypeStruct((N_IN, G_HALF), jnp.float32)]

_CPROJ_OUT_SPECS = [pl.BlockSpec((_R, G_OUT), lambda i: (i, 0)),
                    pl.BlockSpec((_R, G_OUT), lambda i: (i, 0)),
                    pl.BlockSpec((_R, G_HALF), lambda i: (i, 0)),
                    pl.BlockSpec((_R, G_HALF), lambda i: (i, 0))]


def _cproj(s, ylo, yhi, plo, phi, ws, wlo, whi, b):
    return pl.pallas_call(
        _cproj_body,
        grid=(_NG,),
        in_specs=[_row_spec(G_OUT), _row_spec(G_HALF), _row_spec(G_HALF),
                  _full_spec(G_HALF, G_OUT), _full_spec(G_HALF, G_OUT),
                  _full_spec(G_OUT, G_OUT), _full_spec(G_OUT, G_HALF),
                  _full_spec(G_OUT, G_HALF), _full_spec(1, G_OUT)],
        out_specs=_CPROJ_OUT_SPECS,
        out_shape=_CPROJ_OUT,
    )(s, ylo, yhi, plo, phi, ws, wlo, whi, b)


def _cproj_res(res, s, ylo, yhi, plo, phi, ws, wlo, whi, b):
    return pl.pallas_call(
        _cproj_res_body,
        grid=(_NG,),
        in_specs=[_row_spec(G_OUT),
                  _row_spec(G_OUT), _row_spec(G_HALF), _row_spec(G_HALF),
                  _full_spec(G_HALF, G_OUT), _full_spec(G_HALF, G_OUT),
                  _full_spec(G_OUT, G_OUT), _full_spec(G_OUT, G_HALF),
                  _full_spec(G_OUT, G_HALF), _full_spec(1, G_OUT)],
        out_specs=_CPROJ_OUT_SPECS,
        out_shape=_CPROJ_OUT,
    )(res, s, ylo, yhi, plo, phi, ws, wlo, whi, b)


def _ccls_body(res_ref, s_ref, ylo_ref, yhi_ref, plo_ref, phi_ref,
               wcs_ref, wcn_ref, bc_ref,
               h_ref, sc_ref, tpad_ref):
    y = (jnp.dot(ylo_ref[...], plo_ref[...], preferred_element_type=jnp.float32)
         + jnp.dot(yhi_ref[...], phi_ref[...], preferred_element_type=jnp.float32))
    h = jnp.maximum(res_ref[...] + s_ref[...] + y, 0.0)
    h_ref[...] = h
    sc_ref[...] = (jnp.dot(h, wcs_ref[...], preferred_element_type=jnp.float32)
                   + bc_ref[...])
    tpad_ref[...] = jnp.dot(h, wcn_ref[...], preferred_element_type=jnp.float32)


def _ccls(res, s, ylo, yhi, plo, phi, wcs, wcn, bc):
    return pl.pallas_call(
        _ccls_body,
        grid=(_NG,),
        in_specs=[_row_spec(G_OUT),
                  _row_spec(G_OUT), _row_spec(G_HALF), _row_spec(G_HALF),
                  _full_spec(G_HALF, G_OUT), _full_spec(G_HALF, G_OUT),
                  _full_spec(G_OUT, G_HALF), _full_spec(G_OUT, G_HALF),
                  _full_spec(1, G_HALF)],
        out_specs=[pl.BlockSpec((_R, G_OUT), lambda i: (i, 0)),
                   pl.BlockSpec((_R, G_HALF), lambda i: (i, 0)),
                   pl.BlockSpec((_R, G_HALF), lambda i: (i, 0))],
        out_shape=[jax.ShapeDtypeStruct((N_IN, G_OUT), jnp.float32),
                   jax.ShapeDtypeStruct((N_IN, G_HALF), jnp.float32),
                   jax.ShapeDtypeStruct((N_IN, G_HALF), jnp.float32)],
    )(res, s, ylo, yhi, plo, phi, wcs, wcn, bc)


def _finreduce_body(sc_ref, y0_ref, y1_ref, lg_ref, mx_ref, top_ref):
    lg = sc_ref[...] + y0_ref[...] + y1_ref[...]
    lg_ref[...] = lg
    col = lax.broadcasted_iota(jnp.int32, (N_IN, G_HALF), 1)
    node = (lax.broadcasted_iota(jnp.int32, (N_IN, G_HALF), 0) * UP
            + col // HALF)
    valid = (col % HALF) == 0
    neg = jnp.float32(-3.0e38)
    mx = jnp.max(jnp.where(valid, lg, neg))
    mx_ref[...] = jnp.reshape(mx, (1, 1))
    top_ref[...] = jnp.reshape(
        jnp.min(jnp.where(valid & (lg == mx), node, jnp.int32(2**30))), (1, 1))


def _finreduce(sc, y0, y1):
    return pl.pallas_call(
        _finreduce_body,
        in_specs=[pl.BlockSpec((N_IN, G_HALF), lambda: (0, 0))] * 3,
        out_specs=[pl.BlockSpec((N_IN, G_HALF), lambda: (0, 0)),
                   pl.BlockSpec((1, 1), lambda: (0, 0)),
                   pl.BlockSpec((1, 1), lambda: (0, 0))],
        out_shape=[jax.ShapeDtypeStruct((N_IN, G_HALF), jnp.float32),
                   jax.ShapeDtypeStruct((1, 1), jnp.float32),
                   jax.ShapeDtypeStruct((1, 1), jnp.int32)],
    )(sc, y0, y1)


def _mask_body(h_ref, lg_ref, tgt_ref, mx_ref, top_ref, o_ref, k_ref):
    R = 4000
    pid = pl.program_id(0)
    lg = lg_ref[...]
    iota = lax.broadcasted_iota(jnp.int32, (R, 1), 0) + pid * R
    keep = ((lg > 0.0) | (tgt_ref[...] != 0)
            | ((iota == top_ref[0, 0]) & (mx_ref[0, 0] < 0.0)))
    o_ref[...] = h_ref[...] * keep.astype(jnp.float32)
    k_ref[...] = keep.astype(jnp.int32)


def _mask(h, lgN, tgtN, mx, top):
    R = 4000
    return pl.pallas_call(
        _mask_body,
        grid=(N_UP // R,),
        in_specs=[pl.BlockSpec((R, C_OUT), lambda i: (i, 0)),
                  pl.BlockSpec((R, 1), lambda i: (i, 0)),
                  pl.BlockSpec((R, 1), lambda i: (i, 0)),
                  pl.BlockSpec((1, 1), lambda i: (0, 0)),
                  pl.BlockSpec((1, 1), lambda i: (0, 0))],
        out_specs=[pl.BlockSpec((R, C_OUT), lambda i: (i, 0)),
                   pl.BlockSpec((R, 1), lambda i: (i, 0))],
        out_shape=[jax.ShapeDtypeStruct((N_UP, C_OUT), jnp.float32),
                   jax.ShapeDtypeStruct((N_UP, 1), jnp.int32)],
    )(h, lgN, tgtN, mx, top)


_MESH = plsc.VectorSubcoreMesh(core_axis_name="c", subcore_axis_name="s",
                               num_cores=2, num_subcores=NS)

_SC_SCRATCH = (
    [pltpu.VMEM((K_CH,), jnp.int32) for _ in range(NSLOT)]
    + [pltpu.VMEM((K_CH,), jnp.int32) for _ in range(NSLOT)]
    + [pltpu.VMEM((K_CH, HALF), jnp.float32) for _ in range(NSLOT)]
    + [pltpu.VMEM_SHARED((N_UP, HALF), jnp.float32)]
    + [pltpu.SemaphoreType.DMA] * (3 * NSLOT)
)


def _mk_slots(scr):
    srcvs = scr[0:NSLOT]
    dstvs = scr[NSLOT:2 * NSLOT]
    rows = scr[2 * NSLOT:3 * NSLOT]
    sems = scr[3 * NSLOT + 1:]
    isems, gsems, ssems = (sems[0:NSLOT], sems[NSLOT:2 * NSLOT],
                           sems[2 * NSLOT:3 * NSLOT])
    return [(srcvs[s], dstvs[s], rows[s], isems[s], gsems[s], ssems[s])
            for s in range(NSLOT)]


def _edge_loop(table_h, src_h, dst_h, acc, slots, base0, nch):
    def idx_copies(slot, c):
        srcv, dstv, _, isem, _, _ = slots[slot]
        base = base0 + c * K_CH
        return (pltpu.make_async_copy(src_h.at[pl.ds(base, K_CH)], srcv, isem),
                pltpu.make_async_copy(dst_h.at[pl.ds(base, K_CH)], dstv, isem))

    def stage(c_idx, s_idx, c_gat, s_gat, c_sct, s_sct):
        srcv, dstv, rows, isem, gsem, ssem = slots[s_idx]

        @pl.when((c_idx < nch) & (c_idx >= NSLOT))
        def _():
            pltpu.make_async_copy(rows, acc.at[dstv], ssem).wait()

        @pl.when(c_idx < nch)
        def _():
            for d in idx_copies(s_idx, c_idx):
                d.start()

        srcvg, dstvg, rowsg, isemg, gsemg, ssemg = slots[s_gat]

        @pl.when((c_gat >= 0) & (c_gat < nch))
        def _():
            for d in idx_copies(s_gat, c_gat):
                d.wait()
            pltpu.async_copy(table_h.at[srcvg], rowsg, gsemg)

        srcvs, dstvs, rowss, isems_, gsems_, ssems_ = slots[s_sct]

        @pl.when((c_sct >= 0) & (c_sct < nch))
        def _():
            pltpu.make_async_copy(table_h.at[srcvs], rowss, gsems_).wait()
            pltpu.async_copy(rowss, acc.at[dstvs], ssems_, add=True)

    nsup = (nch + 2 + NSLOT - 1) // NSLOT

    def sup(i, carry):
        g0 = i * NSLOT
        for j in range(NSLOT):
            g = g0 + j
            stage(g, j, g - 1, (j - 1) % NSLOT, g - 2, (j - 2) % NSLOT)
        return carry
    lax.fori_loop(0, nsup, sup, 0)
    for c in range(max(0, nch - NSLOT), nch):
        _, dstvs, rowss, _, _, ssems_ = slots[c % NSLOT]
        pltpu.make_async_copy(rowss, acc.at[dstvs], ssems_).wait()


@functools.partial(
    pl.kernel,
    out_type=[jax.ShapeDtypeStruct((N_UP, HALF), jnp.float32)] * 2,
    mesh=_MESH,
    scratch_types=_SC_SCRATCH,
    compiler_params=pltpu.CompilerParams(use_tc_tiling_on_sc=False),
)
def _sc_wide(tlo_h, thi_h, src_h, dst_h, zer_h, ylo_h, yhi_h, *scr):
    cid = lax.axis_index("c")
    sid = lax.axis_index("s")
    slots = _mk_slots(scr)
    acc = scr[3 * NSLOT]
    rpt = N_UP // NS
    row0 = sid * rpt
    ept = E_UP // NS
    nch = ept // K_CH
    pltpu.sync_copy(zer_h.at[pl.ds(row0, rpt)], acc.at[pl.ds(row0, rpt)])
    plsc.subcore_barrier()

    @pl.when(cid == 0)
    def _():
        _edge_loop(tlo_h, src_h, dst_h, acc, slots, sid * ept, nch)

    @pl.when(cid == 1)
    def _():
        _edge_loop(thi_h, src_h, dst_h, acc, slots, sid * ept, nch)

    plsc.subcore_barrier()

    @pl.when(cid == 0)
    def _():
        pltpu.sync_copy(acc.at[pl.ds(row0, rpt)], ylo_h.at[pl.ds(row0, rpt)])

    @pl.when(cid == 1)
    def _():
        pltpu.sync_copy(acc.at[pl.ds(row0, rpt)], yhi_h.at[pl.ds(row0, rpt)])


@functools.partial(
    pl.kernel,
    out_type=[jax.ShapeDtypeStruct((N_UP, HALF), jnp.float32)] * 2,
    mesh=_MESH,
    scratch_types=_SC_SCRATCH,
    compiler_params=pltpu.CompilerParams(use_tc_tiling_on_sc=False),
)
def _sc_cls(t_h, src_h, dst_h, zer_h, y0_h, y1_h, *scr):
    cid = lax.axis_index("c")
    sid = lax.axis_index("s")
    slots = _mk_slots(scr)
    acc = scr[3 * NSLOT]
    rpt = N_UP // NS
    row0 = sid * rpt
    ept = E_UP // (2 * NS)
    nch = ept // K_CH
    base0 = cid * (E_UP // 2) + sid * ept
    pltpu.sync_copy(zer_h.at[pl.ds(row0, rpt)], acc.at[pl.ds(row0, rpt)])
    plsc.subcore_barrier()
    _edge_loop(t_h, src_h, dst_h, acc, slots, base0, nch)
    plsc.subcore_barrier()

    @pl.when(cid == 0)
    def _():
        pltpu.sync_copy(acc.at[pl.ds(row0, rpt)], y0_h.at[pl.ds(row0, rpt)])

    @pl.when(cid == 1)
    def _():
        pltpu.sync_copy(acc.at[pl.ds(row0, rpt)], y1_h.at[pl.ds(row0, rpt)])


def _bd8(w):
    c, d = w.shape
    return jnp.einsum('jk,cd->jckd', jnp.eye(UP, dtype=w.dtype), w).reshape(
        UP * c, UP * d)


def _grouped(a):
    return a.reshape(N_IN, G_HALF)


def _flat16(a):
    return a.reshape(N_UP, HALF)


def kernel(x, edge_index_up, target_label, W_up, b_up, W1_self, W1_nbr, b1,
           Wb_self, Wb_nbr, bb, Wc_self, Wc_nbr, bc):
    f32 = jnp.float32
    src = edge_index_up[0]
    dst = edge_index_up[1]
    zer = jnp.zeros((N_UP, HALF), f32)

    plo = _bd8(jnp.concatenate(
        [jnp.eye(HALF, dtype=f32), jnp.zeros((HALF, HALF), f32)], axis=1))
    phi = _bd8(jnp.concatenate(
        [jnp.zeros((HALF, HALF), f32), jnp.eye(HALF, dtype=f32)], axis=1))

    wf_up = jnp.transpose(W_up, (1, 0, 2)).reshape(C_IN, G_HID)
    bf_up = jnp.tile(b_up, UP).reshape(1, G_HID)
    h = _up(x, wf_up, bf_up)

    s, tlo, thi = _proj0(h, _bd8(W1_self), _bd8(W1_nbr[:, :HALF]),
                         _bd8(W1_nbr[:, HALF:]),
                         jnp.tile(b1, UP).reshape(1, G_OUT))
    ylo, yhi = _sc_wide(_flat16(tlo), _flat16(thi), src, dst, zer)

    hres, s, tlo, thi = _cproj(
        s, _grouped(ylo), _grouped(yhi), plo, phi,
        _bd8(Wb_self[0, 0]), _bd8(Wb_nbr[0, 0][:, :HALF]),
        _bd8(Wb_nbr[0, 0][:, HALF:]), jnp.tile(bb[0, 0], UP).reshape(1, G_OUT))
    ylo, yhi = _sc_wide(_flat16(tlo), _flat16(thi), src, dst, zer)

    for l in range(L_BLOCK):
        _, s, tlo, thi = _cproj(
            s, _grouped(ylo), _grouped(yhi), plo, phi,
            _bd8(Wb_self[l, 1]), _bd8(Wb_nbr[l, 1][:, :HALF]),
            _bd8(Wb_nbr[l, 1][:, HALF:]),
            jnp.tile(bb[l, 1], UP).reshape(1, G_OUT))
        ylo, yhi = _sc_wide(_flat16(tlo), _flat16(thi), src, dst, zer)
        if l < L_BLOCK - 1:
            hres, s, tlo, thi = _cproj_res(
                hres, s, _grouped(ylo), _grouped(yhi), plo, phi,
                _bd8(Wb_self[l + 1, 0]), _bd8(Wb_nbr[l + 1, 0][:, :HALF]),
                _bd8(Wb_nbr[l + 1, 0][:, HALF:]),
                jnp.tile(bb[l + 1, 0], UP).reshape(1, G_OUT))
            ylo, yhi = _sc_wide(_flat16(tlo), _flat16(thi), src, dst, zer)

    e0 = jnp.zeros((1, HALF), f32).at[0, 0].set(1.0)
    wcs = _bd8(Wc_self @ e0)
    wcn = _bd8(Wc_nbr @ e0)
    bc_g = jnp.tile(bc[0] * e0, (1, UP))
    hfin, sc_g, tpad = _ccls(hres, s, _grouped(ylo), _grouped(yhi), plo, phi,
                             wcs, wcn, bc_g)
    y0, y1 = _sc_cls(_flat16(tpad), src, dst, zer)

    lg_g, mx, top = _finreduce(sc_g, _grouped(y0), _grouped(y1))

    out_cls = lg_g.reshape(N_UP, HALF)[:, :1]
    hN = hfin.reshape(N_UP, C_OUT)
    tgtN = target_label.astype(jnp.int32).reshape(N_UP, 1)
    out_pruned, keep_i = _mask(hN, out_cls, tgtN, mx, top)
    keep = keep_i.reshape(N_UP) != 0
    return out_pruned, out_cls, target_label, keep


# speedup vs baseline: 1.0742x; 1.0094x over previous
import functools

import jax
import jax.numpy as jnp
from jax import lax
from jax.experimental import pallas as pl
from jax.experimental.pallas import tpu as pltpu
from jax.experimental.pallas import tpu_sc as plsc

N_IN = 10000
UP = 8
N_UP = N_IN * UP
E_UP = 1280000
C_IN = 64
C_HID = 64
C_OUT = 32
HALF = 16
L_BLOCK = 3

G_HID = UP * C_HID
G_OUT = UP * C_OUT
G_HALF = UP * HALF

NS = 16
K_CH = 800
NSLOT = 3

_R = 2000
_NG = N_IN // _R


def _upproj_body(x_ref, wu_ref, bu_ref, ws_ref, wlo_ref, whi_ref, b_ref,
                 s_ref, tlo_ref, thi_ref):
    h = jnp.maximum(
        jnp.dot(x_ref[...], wu_ref[...], preferred_element_type=jnp.float32)
        + bu_ref[...], 0.0)
    s_ref[...] = (jnp.dot(h, ws_ref[...], preferred_element_type=jnp.float32)
                  + b_ref[...])
    tlo_ref[...] = jnp.dot(h, wlo_ref[...], preferred_element_type=jnp.float32)
    thi_ref[...] = jnp.dot(h, whi_ref[...], preferred_element_type=jnp.float32)


def _upproj(x, wu, bu, ws, wlo, whi, b):
    return pl.pallas_call(
        _upproj_body,
        grid=(_NG,),
        in_specs=[pl.BlockSpec((_R, C_IN), lambda i: (i, 0)),
                  pl.BlockSpec((C_IN, G_HID), lambda i: (0, 0)),
                  pl.BlockSpec((1, G_HID), lambda i: (0, 0)),
                  pl.BlockSpec((G_HID, G_OUT), lambda i: (0, 0)),
                  pl.BlockSpec((G_HID, G_HALF), lambda i: (0, 0)),
                  pl.BlockSpec((G_HID, G_HALF), lambda i: (0, 0)),
                  pl.BlockSpec((1, G_OUT), lambda i: (0, 0))],
        out_specs=[pl.BlockSpec((_R, G_OUT), lambda i: (i, 0)),
                   pl.BlockSpec((_R, G_HALF), lambda i: (i, 0)),
                   pl.BlockSpec((_R, G_HALF), lambda i: (i, 0))],
        out_shape=[jax.ShapeDtypeStruct((N_IN, G_OUT), jnp.float32),
                   jax.ShapeDtypeStruct((N_IN, G_HALF), jnp.float32),
                   jax.ShapeDtypeStruct((N_IN, G_HALF), jnp.float32)],
    )(x, wu, bu, ws, wlo, whi, b)


def _cproj_body(s_ref, ylo_ref, yhi_ref, plo_ref, phi_ref,
                ws_ref, wlo_ref, whi_ref, b_ref,
                h_ref, s2_ref, tlo_ref, thi_ref):
    y = (jnp.dot(ylo_ref[...], plo_ref[...], preferred_element_type=jnp.float32)
         + jnp.dot(yhi_ref[...], phi_ref[...], preferred_element_type=jnp.float32))
    h = jnp.maximum(s_ref[...] + y, 0.0)
    h_ref[...] = h
    s2_ref[...] = (jnp.dot(h, ws_ref[...], preferred_element_type=jnp.float32)
                   + b_ref[...])
    tlo_ref[...] = jnp.dot(h, wlo_ref[...], preferred_element_type=jnp.float32)
    thi_ref[...] = jnp.dot(h, whi_ref[...], preferred_element_type=jnp.float32)


def _cproj_res_body(res_ref, s_ref, ylo_ref, yhi_ref, plo_ref, phi_ref,
                    ws_ref, wlo_ref, whi_ref, b_ref,
                    h_ref, s2_ref, tlo_ref, thi_ref):
    y = (jnp.dot(ylo_ref[...], plo_ref[...], preferred_element_type=jnp.float32)
         + jnp.dot(yhi_ref[...], phi_ref[...], preferred_element_type=jnp.float32))
    h = jnp.maximum(res_ref[...] + s_ref[...] + y, 0.0)
    h_ref[...] = h
    s2_ref[...] = (jnp.dot(h, ws_ref[...], preferred_element_type=jnp.float32)
                   + b_ref[...])
    tlo_ref[...] = jnp.dot(h, wlo_ref[...], preferred_element_type=jnp.float32)
    thi_ref[...] = jnp.dot(h, whi_ref[...], preferred_element_type=jnp.float32)


def _row_spec(w):
    return pl.BlockSpec((_R, w), lambda i: (i, 0))


def _full_spec(r, w):
    return pl.BlockSpec((r, w), lambda i: (0, 0))


_CPROJ_OUT = [jax.ShapeDtypeStruct((N_IN, G_OUT), jnp.float32),
              jax.ShapeDtypeStruct((N_IN, G_OUT), jnp.float32),
              jax.ShapeDtypeStruct((N_IN, G_HALF), jnp.float32),
              jax.ShapeDtypeStruct((N_IN, G_HALF), jnp.float32)]

_CPROJ_OUT_SPECS = [pl.BlockSpec((_R, G_OUT), lambda i: (i, 0)),
                    pl.BlockSpec((_R, G_OUT), lambda i: (i, 0)),
                    pl.BlockSpec((_R, G_HALF), lambda i: (i, 0)),
                    pl.BlockSpec((_R, G_HALF), lambda i: (i, 0))]


def _cproj(s, ylo, yhi, plo, phi, ws, wlo, whi, b):
    return pl.pallas_call(
        _cproj_body,
        grid=(_NG,),
        in_specs=[_row_spec(G_OUT), _row_spec(G_HALF), _row_spec(G_HALF),
                  _full_spec(G_HALF, G_OUT), _full_spec(G_HALF, G_OUT),
                  _full_spec(G_OUT, G_OUT), _full_spec(G_OUT, G_HALF),
                  _full_spec(G_OUT, G_HALF), _full_spec(1, G_OUT)],
        out_specs=_CPROJ_OUT_SPECS,
        out_shape=_CPROJ_OUT,
    )(s, ylo, yhi, plo, phi, ws, wlo, whi, b)


def _cproj_res(res, s, ylo, yhi, plo, phi, ws, wlo, whi, b):
    return pl.pallas_call(
        _cproj_res_body,
        grid=(_NG,),
        in_specs=[_row_spec(G_OUT),
                  _row_spec(G_OUT), _row_spec(G_HALF), _row_spec(G_HALF),
                  _full_spec(G_HALF, G_OUT), _full_spec(G_HALF, G_OUT),
                  _full_spec(G_OUT, G_OUT), _full_spec(G_OUT, G_HALF),
                  _full_spec(G_OUT, G_HALF), _full_spec(1, G_OUT)],
        out_specs=_CPROJ_OUT_SPECS,
        out_shape=_CPROJ_OUT,
    )(res, s, ylo, yhi, plo, phi, ws, wlo, whi, b)


def _ccls_body(res_ref, s_ref, ylo_ref, yhi_ref, plo_ref, phi_ref,
               wcs_ref, wcn_ref, bc_ref,
               h_ref, sc_ref, tpad_ref):
    y = (jnp.dot(ylo_ref[...], plo_ref[...], preferred_element_type=jnp.float32)
         + jnp.dot(yhi_ref[...], phi_ref[...], preferred_element_type=jnp.float32))
    h = jnp.maximum(res_ref[...] + s_ref[...] + y, 0.0)
    h_ref[...] = h
    sc_ref[...] = (jnp.dot(h, wcs_ref[...], preferred_element_type=jnp.float32)
                   + bc_ref[...])
    tpad_ref[...] = jnp.dot(h, wcn_ref[...], preferred_element_type=jnp.float32)


def _ccls(res, s, ylo, yhi, plo, phi, wcs, wcn, bc):
    return pl.pallas_call(
        _ccls_body,
        grid=(_NG,),
        in_specs=[_row_spec(G_OUT),
                  _row_spec(G_OUT), _row_spec(G_HALF), _row_spec(G_HALF),
                  _full_spec(G_HALF, G_OUT), _full_spec(G_HALF, G_OUT),
                  _full_spec(G_OUT, G_HALF), _full_spec(G_OUT, G_HALF),
                  _full_spec(1, G_HALF)],
        out_specs=[pl.BlockSpec((_R, G_OUT), lambda i: (i, 0)),
                   pl.BlockSpec((_R, G_HALF), lambda i: (i, 0)),
                   pl.BlockSpec((_R, G_HALF), lambda i: (i, 0))],
        out_shape=[jax.ShapeDtypeStruct((N_IN, G_OUT), jnp.float32),
                   jax.ShapeDtypeStruct((N_IN, G_HALF), jnp.float32),
                   jax.ShapeDtypeStruct((N_IN, G_HALF), jnp.float32)],
    )(res, s, ylo, yhi, plo, phi, wcs, wcn, bc)


def _finreduce_body(sc_ref, y0_ref, y1_ref, lg_ref, mx_ref, top_ref):
    lg = sc_ref[...] + y0_ref[...] + y1_ref[...]
    lg_ref[...] = lg
    col = lax.broadcasted_iota(jnp.int32, (N_IN, G_HALF), 1)
    node = (lax.broadcasted_iota(jnp.int32, (N_IN, G_HALF), 0) * UP
            + col // HALF)
    valid = (col % HALF) == 0
    neg = jnp.float32(-3.0e38)
    mx = jnp.max(jnp.where(valid, lg, neg))
    mx_ref[...] = jnp.reshape(mx, (1, 1))
    top_ref[...] = jnp.reshape(
        jnp.min(jnp.where(valid & (lg == mx), node, jnp.int32(2**30))), (1, 1))


def _finreduce(sc, y0, y1):
    return pl.pallas_call(
        _finreduce_body,
        in_specs=[pl.BlockSpec((N_IN, G_HALF), lambda: (0, 0))] * 3,
        out_specs=[pl.BlockSpec((N_IN, G_HALF), lambda: (0, 0)),
                   pl.BlockSpec((1, 1), lambda: (0, 0)),
                   pl.BlockSpec((1, 1), lambda: (0, 0))],
        out_shape=[jax.ShapeDtypeStruct((N_IN, G_HALF), jnp.float32),
                   jax.ShapeDtypeStruct((1, 1), jnp.float32),
                   jax.ShapeDtypeStruct((1, 1), jnp.int32)],
    )(sc, y0, y1)


def _mask_body(h_ref, lg_ref, tgt_ref, mx_ref, top_ref, o_ref, k_ref):
    R = 4000
    pid = pl.program_id(0)
    lg = lg_ref[...]
    iota = lax.broadcasted_iota(jnp.int32, (R, 1), 0) + pid * R
    keep = ((lg > 0.0) | (tgt_ref[...] != 0)
            | ((iota == top_ref[0, 0]) & (mx_ref[0, 0] < 0.0)))
    o_ref[...] = h_ref[...] * keep.astype(jnp.float32)
    k_ref[...] = keep.astype(jnp.int32)


def _mask(h, lgN, tgtN, mx, top):
    R = 4000
    return pl.pallas_call(
        _mask_body,
        grid=(N_UP // R,),
        in_specs=[pl.BlockSpec((R, C_OUT), lambda i: (i, 0)),
                  pl.BlockSpec((R, 1), lambda i: (i, 0)),
                  pl.BlockSpec((R, 1), lambda i: (i, 0)),
                  pl.BlockSpec((1, 1), lambda i: (0, 0)),
                  pl.BlockSpec((1, 1), lambda i: (0, 0))],
        out_specs=[pl.BlockSpec((R, C_OUT), lambda i: (i, 0)),
                   pl.BlockSpec((R, 1), lambda i: (i, 0))],
        out_shape=[jax.ShapeDtypeStruct((N_UP, C_OUT), jnp.float32),
                   jax.ShapeDtypeStruct((N_UP, 1), jnp.int32)],
    )(h, lgN, tgtN, mx, top)


_MESH = plsc.VectorSubcoreMesh(core_axis_name="c", subcore_axis_name="s",
                               num_cores=2, num_subcores=NS)

_SC_SCRATCH = (
    [pltpu.VMEM((K_CH,), jnp.int32) for _ in range(NSLOT)]
    + [pltpu.VMEM((K_CH,), jnp.int32) for _ in range(NSLOT)]
    + [pltpu.VMEM((K_CH, HALF), jnp.float32) for _ in range(NSLOT)]
    + [pltpu.VMEM_SHARED((N_UP, HALF), jnp.float32)]
    + [pltpu.SemaphoreType.DMA] * (3 * NSLOT)
)


def _mk_slots(scr):
    srcvs = scr[0:NSLOT]
    dstvs = scr[NSLOT:2 * NSLOT]
    rows = scr[2 * NSLOT:3 * NSLOT]
    sems = scr[3 * NSLOT + 1:]
    isems, gsems, ssems = (sems[0:NSLOT], sems[NSLOT:2 * NSLOT],
                           sems[2 * NSLOT:3 * NSLOT])
    return [(srcvs[s], dstvs[s], rows[s], isems[s], gsems[s], ssems[s])
            for s in range(NSLOT)]


def _edge_loop(table_h, src_h, dst_h, acc, slots, base0, nch):
    def idx_copies(slot, c):
        srcv, dstv, _, isem, _, _ = slots[slot]
        base = base0 + c * K_CH
        return (pltpu.make_async_copy(src_h.at[pl.ds(base, K_CH)], srcv, isem),
                pltpu.make_async_copy(dst_h.at[pl.ds(base, K_CH)], dstv, isem))

    def stage(c_idx, s_idx, c_gat, s_gat, c_sct, s_sct):
        srcv, dstv, rows, isem, gsem, ssem = slots[s_idx]

        @pl.when((c_idx < nch) & (c_idx >= NSLOT))
        def _():
            pltpu.make_async_copy(rows, acc.at[dstv], ssem).wait()

        @pl.when(c_idx < nch)
        def _():
            for d in idx_copies(s_idx, c_idx):
                d.start()

        srcvg, dstvg, rowsg, isemg, gsemg, ssemg = slots[s_gat]

        @pl.when((c_gat >= 0) & (c_gat < nch))
        def _():
            for d in idx_copies(s_gat, c_gat):
                d.wait()
            pltpu.async_copy(table_h.at[srcvg], rowsg, gsemg)

        srcvs, dstvs, rowss, isems_, gsems_, ssems_ = slots[s_sct]

        @pl.when((c_sct >= 0) & (c_sct < nch))
        def _():
            pltpu.make_async_copy(table_h.at[srcvs], rowss, gsems_).wait()
            pltpu.async_copy(rowss, acc.at[dstvs], ssems_, add=True)

    nsup = (nch + 2 + NSLOT - 1) // NSLOT

    def sup(i, carry):
        g0 = i * NSLOT
        for j in range(NSLOT):
            g = g0 + j
            stage(g, j, g - 1, (j - 1) % NSLOT, g - 2, (j - 2) % NSLOT)
        return carry
    lax.fori_loop(0, nsup, sup, 0)
    for c in range(max(0, nch - NSLOT), nch):
        _, dstvs, rowss, _, _, ssems_ = slots[c % NSLOT]
        pltpu.make_async_copy(rowss, acc.at[dstvs], ssems_).wait()


@functools.partial(
    pl.kernel,
    out_type=[jax.ShapeDtypeStruct((N_UP, HALF), jnp.float32)] * 2,
    mesh=_MESH,
    scratch_types=_SC_SCRATCH,
    compiler_params=pltpu.CompilerParams(use_tc_tiling_on_sc=False),
)
def _sc_wide(tlo_h, thi_h, src_h, dst_h, zer_h, ylo_h, yhi_h, *scr):
    cid = lax.axis_index("c")
    sid = lax.axis_index("s")
    slots = _mk_slots(scr)
    acc = scr[3 * NSLOT]
    rpt = N_UP // NS
    row0 = sid * rpt
    ept = E_UP // NS
    nch = ept // K_CH
    pltpu.sync_copy(zer_h.at[pl.ds(row0, rpt)], acc.at[pl.ds(row0, rpt)])
    plsc.subcore_barrier()

    @pl.when(cid == 0)
    def _():
        _edge_loop(tlo_h, src_h, dst_h, acc, slots, sid * ept, nch)

    @pl.when(cid == 1)
    def _():
        _edge_loop(thi_h, src_h, dst_h, acc, slots, sid * ept, nch)

    plsc.subcore_barrier()

    @pl.when(cid == 0)
    def _():
        pltpu.sync_copy(acc.at[pl.ds(row0, rpt)], ylo_h.at[pl.ds(row0, rpt)])

    @pl.when(cid == 1)
    def _():
        pltpu.sync_copy(acc.at[pl.ds(row0, rpt)], yhi_h.at[pl.ds(row0, rpt)])


@functools.partial(
    pl.kernel,
    out_type=[jax.ShapeDtypeStruct((N_UP, HALF), jnp.float32)] * 2,
    mesh=_MESH,
    scratch_types=_SC_SCRATCH,
    compiler_params=pltpu.CompilerParams(use_tc_tiling_on_sc=False),
)
def _sc_cls(t_h, src_h, dst_h, zer_h, y0_h, y1_h, *scr):
    cid = lax.axis_index("c")
    sid = lax.axis_index("s")
    slots = _mk_slots(scr)
    acc = scr[3 * NSLOT]
    rpt = N_UP // NS
    row0 = sid * rpt
    ept = E_UP // (2 * NS)
    nch = ept // K_CH
    base0 = cid * (E_UP // 2) + sid * ept
    pltpu.sync_copy(zer_h.at[pl.ds(row0, rpt)], acc.at[pl.ds(row0, rpt)])
    plsc.subcore_barrier()
    _edge_loop(t_h, src_h, dst_h, acc, slots, base0, nch)
    plsc.subcore_barrier()

    @pl.when(cid == 0)
    def _():
        pltpu.sync_copy(acc.at[pl.ds(row0, rpt)], y0_h.at[pl.ds(row0, rpt)])

    @pl.when(cid == 1)
    def _():
        pltpu.sync_copy(acc.at[pl.ds(row0, rpt)], y1_h.at[pl.ds(row0, rpt)])


def _bd8(w):
    c, d = w.shape
    return jnp.einsum('jk,cd->jckd', jnp.eye(UP, dtype=w.dtype), w).reshape(
        UP * c, UP * d)


def _grouped(a):
    return a.reshape(N_IN, G_HALF)


def _flat16(a):
    return a.reshape(N_UP, HALF)


def kernel(x, edge_index_up, target_label, W_up, b_up, W1_self, W1_nbr, b1,
           Wb_self, Wb_nbr, bb, Wc_self, Wc_nbr, bc):
    f32 = jnp.float32
    src = edge_index_up[0]
    dst = edge_index_up[1]
    zer = jnp.zeros((N_UP, HALF), f32)

    plo = _bd8(jnp.concatenate(
        [jnp.eye(HALF, dtype=f32), jnp.zeros((HALF, HALF), f32)], axis=1))
    phi = _bd8(jnp.concatenate(
        [jnp.zeros((HALF, HALF), f32), jnp.eye(HALF, dtype=f32)], axis=1))

    wf_up = jnp.transpose(W_up, (1, 0, 2)).reshape(C_IN, G_HID)
    bf_up = jnp.tile(b_up, UP).reshape(1, G_HID)

    s, tlo, thi = _upproj(x, wf_up, bf_up, _bd8(W1_self),
                          _bd8(W1_nbr[:, :HALF]), _bd8(W1_nbr[:, HALF:]),
                          jnp.tile(b1, UP).reshape(1, G_OUT))
    ylo, yhi = _sc_wide(_flat16(tlo), _flat16(thi), src, dst, zer)

    hres, s, tlo, thi = _cproj(
        s, _grouped(ylo), _grouped(yhi), plo, phi,
        _bd8(Wb_self[0, 0]), _bd8(Wb_nbr[0, 0][:, :HALF]),
        _bd8(Wb_nbr[0, 0][:, HALF:]), jnp.tile(bb[0, 0], UP).reshape(1, G_OUT))
    ylo, yhi = _sc_wide(_flat16(tlo), _flat16(thi), src, dst, zer)

    for l in range(L_BLOCK):
        _, s, tlo, thi = _cproj(
            s, _grouped(ylo), _grouped(yhi), plo, phi,
            _bd8(Wb_self[l, 1]), _bd8(Wb_nbr[l, 1][:, :HALF]),
            _bd8(Wb_nbr[l, 1][:, HALF:]),
            jnp.tile(bb[l, 1], UP).reshape(1, G_OUT))
        ylo, yhi = _sc_wide(_flat16(tlo), _flat16(thi), src, dst, zer)
        if l < L_BLOCK - 1:
            hres, s, tlo, thi = _cproj_res(
                hres, s, _grouped(ylo), _grouped(yhi), plo, phi,
                _bd8(Wb_self[l + 1, 0]), _bd8(Wb_nbr[l + 1, 0][:, :HALF]),
                _bd8(Wb_nbr[l + 1, 0][:, HALF:]),
                jnp.tile(bb[l + 1, 0], UP).reshape(1, G_OUT))
            ylo, yhi = _sc_wide(_flat16(tlo), _flat16(thi), src, dst, zer)

    e0 = jnp.zeros((1, HALF), f32).at[0, 0].set(1.0)
    wcs = _bd8(Wc_self @ e0)
    wcn = _bd8(Wc_nbr @ e0)
    bc_g = jnp.tile(bc[0] * e0, (1, UP))
    hfin, sc_g, tpad = _ccls(hres, s, _grouped(ylo), _grouped(yhi), plo, phi,
                             wcs, wcn, bc_g)
    y0, y1 = _sc_cls(_flat16(tpad), src, dst, zer)

    lg_g, mx, top = _finreduce(sc_g, _grouped(y0), _grouped(y1))

    out_cls = lg_g.reshape(N_UP, HALF)[:, :1]
    hN = hfin.reshape(N_UP, C_OUT)
    tgtN = target_label.astype(jnp.int32).reshape(N_UP, 1)
    out_pruned, keep_i = _mask(hN, out_cls, tgtN, mx, top)
    keep = keep_i.reshape(N_UP) != 0
    return out_pruned, out_cls, target_label, keep


# speedup vs baseline: 1.1063x; 1.0299x over previous
import functools

import jax
import jax.numpy as jnp
from jax import lax
from jax.experimental import pallas as pl
from jax.experimental.pallas import tpu as pltpu
from jax.experimental.pallas import tpu_sc as plsc

N_IN = 10000
UP = 8
N_UP = N_IN * UP
E_UP = 1280000
C_IN = 64
C_HID = 64
C_OUT = 32
HALF = 16
L_BLOCK = 3

G_HID = UP * C_HID
G_OUT = UP * C_OUT
G_HALF = UP * HALF

NS = 16
K_CH = 800
NSLOT = 3

_R = 2000
_NG = N_IN // _R


def _upproj_body(x_ref, wu_ref, bu_ref, ws_ref, wlo_ref, whi_ref, b_ref,
                 s_ref, tlo_ref, thi_ref):
    h = jnp.maximum(
        jnp.dot(x_ref[...], wu_ref[...], preferred_element_type=jnp.float32)
        + bu_ref[...], 0.0)
    s_ref[...] = (jnp.dot(h, ws_ref[...], preferred_element_type=jnp.float32)
                  + b_ref[...])
    tlo_ref[...] = jnp.dot(h, wlo_ref[...], preferred_element_type=jnp.float32)
    thi_ref[...] = jnp.dot(h, whi_ref[...], preferred_element_type=jnp.float32)


def _upproj(x, wu, bu, ws, wlo, whi, b):
    return pl.pallas_call(
        _upproj_body,
        grid=(_NG,),
        in_specs=[pl.BlockSpec((_R, C_IN), lambda i: (i, 0)),
                  pl.BlockSpec((C_IN, G_HID), lambda i: (0, 0)),
                  pl.BlockSpec((1, G_HID), lambda i: (0, 0)),
                  pl.BlockSpec((G_HID, G_OUT), lambda i: (0, 0)),
                  pl.BlockSpec((G_HID, G_HALF), lambda i: (0, 0)),
                  pl.BlockSpec((G_HID, G_HALF), lambda i: (0, 0)),
                  pl.BlockSpec((1, G_OUT), lambda i: (0, 0))],
        out_specs=[pl.BlockSpec((_R, G_OUT), lambda i: (i, 0)),
                   pl.BlockSpec((_R, G_HALF), lambda i: (i, 0)),
                   pl.BlockSpec((_R, G_HALF), lambda i: (i, 0))],
        out_shape=[jax.ShapeDtypeStruct((N_IN, G_OUT), jnp.float32),
                   jax.ShapeDtypeStruct((N_IN, G_HALF), jnp.float32),
                   jax.ShapeDtypeStruct((N_IN, G_HALF), jnp.float32)],
    )(x, wu, bu, ws, wlo, whi, b)


def _cproj_body(s_ref, ylo_ref, yhi_ref, plo_ref, phi_ref,
                ws_ref, wlo_ref, whi_ref, b_ref,
                h_ref, s2_ref, tlo_ref, thi_ref):
    y = (jnp.dot(ylo_ref[...], plo_ref[...], preferred_element_type=jnp.float32)
         + jnp.dot(yhi_ref[...], phi_ref[...], preferred_element_type=jnp.float32))
    h = jnp.maximum(s_ref[...] + y, 0.0)
    h_ref[...] = h
    s2_ref[...] = (jnp.dot(h, ws_ref[...], preferred_element_type=jnp.float32)
                   + b_ref[...])
    tlo_ref[...] = jnp.dot(h, wlo_ref[...], preferred_element_type=jnp.float32)
    thi_ref[...] = jnp.dot(h, whi_ref[...], preferred_element_type=jnp.float32)


def _cproj_res_body(res_ref, s_ref, ylo_ref, yhi_ref, plo_ref, phi_ref,
                    ws_ref, wlo_ref, whi_ref, b_ref,
                    h_ref, s2_ref, tlo_ref, thi_ref):
    y = (jnp.dot(ylo_ref[...], plo_ref[...], preferred_element_type=jnp.float32)
         + jnp.dot(yhi_ref[...], phi_ref[...], preferred_element_type=jnp.float32))
    h = jnp.maximum(res_ref[...] + s_ref[...] + y, 0.0)
    h_ref[...] = h
    s2_ref[...] = (jnp.dot(h, ws_ref[...], preferred_element_type=jnp.float32)
                   + b_ref[...])
    tlo_ref[...] = jnp.dot(h, wlo_ref[...], preferred_element_type=jnp.float32)
    thi_ref[...] = jnp.dot(h, whi_ref[...], preferred_element_type=jnp.float32)


def _row_spec(w):
    return pl.BlockSpec((_R, w), lambda i: (i, 0))


def _full_spec(r, w):
    return pl.BlockSpec((r, w), lambda i: (0, 0))


_CPROJ_OUT = [jax.ShapeDtypeStruct((N_IN, G_OUT), jnp.float32),
              jax.ShapeDtypeStruct((N_IN, G_OUT), jnp.float32),
              jax.ShapeDtypeStruct((N_IN, G_HALF), jnp.float32),
              jax.ShapeDtypeStruct((N_IN, G_HALF), jnp.float32)]

_CPROJ_OUT_SPECS = [pl.BlockSpec((_R, G_OUT), lambda i: (i, 0)),
                    pl.BlockSpec((_R, G_OUT), lambda i: (i, 0)),
                    pl.BlockSpec((_R, G_HALF), lambda i: (i, 0)),
                    pl.BlockSpec((_R, G_HALF), lambda i: (i, 0))]


def _cproj(s, ylo, yhi, plo, phi, ws, wlo, whi, b):
    return pl.pallas_call(
        _cproj_body,
        grid=(_NG,),
        in_specs=[_row_spec(G_OUT), _row_spec(G_HALF), _row_spec(G_HALF),
                  _full_spec(G_HALF, G_OUT), _full_spec(G_HALF, G_OUT),
                  _full_spec(G_OUT, G_OUT), _full_spec(G_OUT, G_HALF),
                  _full_spec(G_OUT, G_HALF), _full_spec(1, G_OUT)],
        out_specs=_CPROJ_OUT_SPECS,
        out_shape=_CPROJ_OUT,
    )(s, ylo, yhi, plo, phi, ws, wlo, whi, b)


def _cproj_res(res, s, ylo, yhi, plo, phi, ws, wlo, whi, b):
    return pl.pallas_call(
        _cproj_res_body,
        grid=(_NG,),
        in_specs=[_row_spec(G_OUT),
                  _row_spec(G_OUT), _row_spec(G_HALF), _row_spec(G_HALF),
                  _full_spec(G_HALF, G_OUT), _full_spec(G_HALF, G_OUT),
                  _full_spec(G_OUT, G_OUT), _full_spec(G_OUT, G_HALF),
                  _full_spec(G_OUT, G_HALF), _full_spec(1, G_OUT)],
        out_specs=_CPROJ_OUT_SPECS,
        out_shape=_CPROJ_OUT,
    )(res, s, ylo, yhi, plo, phi, ws, wlo, whi, b)


def _ccls_body(res_ref, s_ref, ylo_ref, yhi_ref, plo_ref, phi_ref,
               wcs_ref, wcn_ref, bc_ref,
               h_ref, sc_ref, tpad_ref):
    y = (jnp.dot(ylo_ref[...], plo_ref[...], preferred_element_type=jnp.float32)
         + jnp.dot(yhi_ref[...], phi_ref[...], preferred_element_type=jnp.float32))
    h = jnp.maximum(res_ref[...] + s_ref[...] + y, 0.0)
    h_ref[...] = h
    sc_ref[...] = (jnp.dot(h, wcs_ref[...], preferred_element_type=jnp.float32)
                   + bc_ref[...])
    tpad_ref[...] = jnp.dot(h, wcn_ref[...], preferred_element_type=jnp.float32)


def _ccls(res, s, ylo, yhi, plo, phi, wcs, wcn, bc):
    return pl.pallas_call(
        _ccls_body,
        grid=(_NG,),
        in_specs=[_row_spec(G_OUT),
                  _row_spec(G_OUT), _row_spec(G_HALF), _row_spec(G_HALF),
                  _full_spec(G_HALF, G_OUT), _full_spec(G_HALF, G_OUT),
                  _full_spec(G_OUT, G_HALF), _full_spec(G_OUT, G_HALF),
                  _full_spec(1, G_HALF)],
        out_specs=[pl.BlockSpec((_R, G_OUT), lambda i: (i, 0)),
                   pl.BlockSpec((_R, G_HALF), lambda i: (i, 0)),
                   pl.BlockSpec((_R, G_HALF), lambda i: (i, 0))],
        out_shape=[jax.ShapeDtypeStruct((N_IN, G_OUT), jnp.float32),
                   jax.ShapeDtypeStruct((N_IN, G_HALF), jnp.float32),
                   jax.ShapeDtypeStruct((N_IN, G_HALF), jnp.float32)],
    )(res, s, ylo, yhi, plo, phi, wcs, wcn, bc)


def _finreduce_body(sc_ref, y0_ref, y1_ref, lg_ref, mx_ref, top_ref):
    lg = sc_ref[...] + y0_ref[...] + y1_ref[...]
    lg_ref[...] = lg
    col = lax.broadcasted_iota(jnp.int32, (N_IN, G_HALF), 1)
    node = (lax.broadcasted_iota(jnp.int32, (N_IN, G_HALF), 0) * UP
            + col // HALF)
    valid = (col % HALF) == 0
    neg = jnp.float32(-3.0e38)
    mx = jnp.max(jnp.where(valid, lg, neg))
    mx_ref[...] = jnp.reshape(mx, (1, 1))
    top_ref[...] = jnp.reshape(
        jnp.min(jnp.where(valid & (lg == mx), node, jnp.int32(2**30))), (1, 1))


def _finreduce(sc, y0, y1):
    return pl.pallas_call(
        _finreduce_body,
        in_specs=[pl.BlockSpec((N_IN, G_HALF), lambda: (0, 0))] * 3,
        out_specs=[pl.BlockSpec((N_IN, G_HALF), lambda: (0, 0)),
                   pl.BlockSpec((1, 1), lambda: (0, 0)),
                   pl.BlockSpec((1, 1), lambda: (0, 0))],
        out_shape=[jax.ShapeDtypeStruct((N_IN, G_HALF), jnp.float32),
                   jax.ShapeDtypeStruct((1, 1), jnp.float32),
                   jax.ShapeDtypeStruct((1, 1), jnp.int32)],
    )(sc, y0, y1)


def _mask_body(h_ref, lg_ref, tgt8_ref, p8_ref, e_ref, mx_ref, top_ref,
               o_ref, k_ref):
    pid = pl.program_id(0)
    lg = lg_ref[...]
    tgtg = jnp.dot(tgt8_ref[...], p8_ref[...],
                   preferred_element_type=jnp.float32)
    col = lax.broadcasted_iota(jnp.int32, (_R, G_HALF), 1)
    node = ((lax.broadcasted_iota(jnp.int32, (_R, G_HALF), 0) + pid * _R) * UP
            + col // HALF)
    q0 = (col % HALF) == 0
    keep = q0 & ((lg > 0.0) | (tgtg > 0.5)
                 | ((node == top_ref[0, 0]) & (mx_ref[0, 0] < 0.0)))
    kf = keep.astype(jnp.float32)
    o_ref[...] = h_ref[...] * jnp.dot(kf, e_ref[...],
                                      preferred_element_type=jnp.float32)
    k_ref[...] = keep.astype(jnp.int32)


def _mask(h, lg_g, tgt8, p8, e128, mx, top):
    return pl.pallas_call(
        _mask_body,
        grid=(_NG,),
        in_specs=[_row_spec(G_OUT), _row_spec(G_HALF),
                  pl.BlockSpec((_R, UP), lambda i: (i, 0)),
                  _full_spec(UP, G_HALF), _full_spec(G_HALF, G_OUT),
                  pl.BlockSpec((1, 1), lambda i: (0, 0)),
                  pl.BlockSpec((1, 1), lambda i: (0, 0))],
        out_specs=[pl.BlockSpec((_R, G_OUT), lambda i: (i, 0)),
                   pl.BlockSpec((_R, G_HALF), lambda i: (i, 0))],
        out_shape=[jax.ShapeDtypeStruct((N_IN, G_OUT), jnp.float32),
                   jax.ShapeDtypeStruct((N_IN, G_HALF), jnp.int32)],
    )(h, lg_g, tgt8, p8, e128, mx, top)


_MESH = plsc.VectorSubcoreMesh(core_axis_name="c", subcore_axis_name="s",
                               num_cores=2, num_subcores=NS)

_SC_SCRATCH = (
    [pltpu.VMEM((K_CH,), jnp.int32) for _ in range(NSLOT)]
    + [pltpu.VMEM((K_CH,), jnp.int32) for _ in range(NSLOT)]
    + [pltpu.VMEM((K_CH, HALF), jnp.float32) for _ in range(NSLOT)]
    + [pltpu.VMEM_SHARED((N_UP, HALF), jnp.float32)]
    + [pltpu.SemaphoreType.DMA] * (3 * NSLOT)
)


def _mk_slots(scr):
    srcvs = scr[0:NSLOT]
    dstvs = scr[NSLOT:2 * NSLOT]
    rows = scr[2 * NSLOT:3 * NSLOT]
    sems = scr[3 * NSLOT + 1:]
    isems, gsems, ssems = (sems[0:NSLOT], sems[NSLOT:2 * NSLOT],
                           sems[2 * NSLOT:3 * NSLOT])
    return [(srcvs[s], dstvs[s], rows[s], isems[s], gsems[s], ssems[s])
            for s in range(NSLOT)]


def _edge_loop(table_h, src_h, dst_h, acc, slots, base0, nch):
    def idx_copies(slot, c):
        srcv, dstv, _, isem, _, _ = slots[slot]
        base = base0 + c * K_CH
        return (pltpu.make_async_copy(src_h.at[pl.ds(base, K_CH)], srcv, isem),
                pltpu.make_async_copy(dst_h.at[pl.ds(base, K_CH)], dstv, isem))

    def stage(c_idx, s_idx, c_gat, s_gat, c_sct, s_sct):
        srcv, dstv, rows, isem, gsem, ssem = slots[s_idx]

        @pl.when((c_idx < nch) & (c_idx >= NSLOT))
        def _():
            pltpu.make_async_copy(rows, acc.at[dstv], ssem).wait()

        @pl.when(c_idx < nch)
        def _():
            for d in idx_copies(s_idx, c_idx):
                d.start()

        srcvg, dstvg, rowsg, isemg, gsemg, ssemg = slots[s_gat]

        @pl.when((c_gat >= 0) & (c_gat < nch))
        def _():
            for d in idx_copies(s_gat, c_gat):
                d.wait()
            pltpu.async_copy(table_h.at[srcvg], rowsg, gsemg)

        srcvs, dstvs, rowss, isems_, gsems_, ssems_ = slots[s_sct]

        @pl.when((c_sct >= 0) & (c_sct < nch))
        def _():
            pltpu.make_async_copy(table_h.at[srcvs], rowss, gsems_).wait()
            pltpu.async_copy(rowss, acc.at[dstvs], ssems_, add=True)

    nsup = (nch + 2 + NSLOT - 1) // NSLOT

    def sup(i, carry):
        g0 = i * NSLOT
        for j in range(NSLOT):
            g = g0 + j
            stage(g, j, g - 1, (j - 1) % NSLOT, g - 2, (j - 2) % NSLOT)
        return carry
    lax.fori_loop(0, nsup, sup, 0)
    for c in range(max(0, nch - NSLOT), nch):
        _, dstvs, rowss, _, _, ssems_ = slots[c % NSLOT]
        pltpu.make_async_copy(rowss, acc.at[dstvs], ssems_).wait()


@functools.partial(
    pl.kernel,
    out_type=[jax.ShapeDtypeStruct((N_UP, HALF), jnp.float32)] * 2,
    mesh=_MESH,
    scratch_types=_SC_SCRATCH,
    compiler_params=pltpu.CompilerParams(use_tc_tiling_on_sc=False),
)
def _sc_wide(tlo_h, thi_h, src_h, dst_h, zer_h, ylo_h, yhi_h, *scr):
    cid = lax.axis_index("c")
    sid = lax.axis_index("s")
    slots = _mk_slots(scr)
    acc = scr[3 * NSLOT]
    rpt = N_UP // NS
    row0 = sid * rpt
    ept = E_UP // NS
    nch = ept // K_CH
    pltpu.sync_copy(zer_h.at[pl.ds(row0, rpt)], acc.at[pl.ds(row0, rpt)])
    plsc.subcore_barrier()

    @pl.when(cid == 0)
    def _():
        _edge_loop(tlo_h, src_h, dst_h, acc, slots, sid * ept, nch)

    @pl.when(cid == 1)
    def _():
        _edge_loop(thi_h, src_h, dst_h, acc, slots, sid * ept, nch)

    plsc.subcore_barrier()

    @pl.when(cid == 0)
    def _():
        pltpu.sync_copy(acc.at[pl.ds(row0, rpt)], ylo_h.at[pl.ds(row0, rpt)])

    @pl.when(cid == 1)
    def _():
        pltpu.sync_copy(acc.at[pl.ds(row0, rpt)], yhi_h.at[pl.ds(row0, rpt)])


@functools.partial(
    pl.kernel,
    out_type=[jax.ShapeDtypeStruct((N_UP, HALF), jnp.float32)] * 2,
    mesh=_MESH,
    scratch_types=_SC_SCRATCH,
    compiler_params=pltpu.CompilerParams(use_tc_tiling_on_sc=False),
)
def _sc_cls(t_h, src_h, dst_h, zer_h, y0_h, y1_h, *scr):
    cid = lax.axis_index("c")
    sid = lax.axis_index("s")
    slots = _mk_slots(scr)
    acc = scr[3 * NSLOT]
    rpt = N_UP // NS
    row0 = sid * rpt
    ept = E_UP // (2 * NS)
    nch = ept // K_CH
    base0 = cid * (E_UP // 2) + sid * ept
    pltpu.sync_copy(zer_h.at[pl.ds(row0, rpt)], acc.at[pl.ds(row0, rpt)])
    plsc.subcore_barrier()
    _edge_loop(t_h, src_h, dst_h, acc, slots, base0, nch)
    plsc.subcore_barrier()

    @pl.when(cid == 0)
    def _():
        pltpu.sync_copy(acc.at[pl.ds(row0, rpt)], y0_h.at[pl.ds(row0, rpt)])

    @pl.when(cid == 1)
    def _():
        pltpu.sync_copy(acc.at[pl.ds(row0, rpt)], y1_h.at[pl.ds(row0, rpt)])


def _bd8(w):
    c, d = w.shape
    return jnp.einsum('jk,cd->jckd', jnp.eye(UP, dtype=w.dtype), w).reshape(
        UP * c, UP * d)


def _grouped(a):
    return a.reshape(N_IN, G_HALF)


def _flat16(a):
    return a.reshape(N_UP, HALF)


def kernel(x, edge_index_up, target_label, W_up, b_up, W1_self, W1_nbr, b1,
           Wb_self, Wb_nbr, bb, Wc_self, Wc_nbr, bc):
    f32 = jnp.float32
    src = edge_index_up[0]
    dst = edge_index_up[1]
    zer = jnp.zeros((N_UP, HALF), f32)

    plo = _bd8(jnp.concatenate(
        [jnp.eye(HALF, dtype=f32), jnp.zeros((HALF, HALF), f32)], axis=1))
    phi = _bd8(jnp.concatenate(
        [jnp.zeros((HALF, HALF), f32), jnp.eye(HALF, dtype=f32)], axis=1))

    wf_up = jnp.transpose(W_up, (1, 0, 2)).reshape(C_IN, G_HID)
    bf_up = jnp.tile(b_up, UP).reshape(1, G_HID)

    s, tlo, thi = _upproj(x, wf_up, bf_up, _bd8(W1_self),
                          _bd8(W1_nbr[:, :HALF]), _bd8(W1_nbr[:, HALF:]),
                          jnp.tile(b1, UP).reshape(1, G_OUT))
    ylo, yhi = _sc_wide(_flat16(tlo), _flat16(thi), src, dst, zer)

    hres, s, tlo, thi = _cproj(
        s, _grouped(ylo), _grouped(yhi), plo, phi,
        _bd8(Wb_self[0, 0]), _bd8(Wb_nbr[0, 0][:, :HALF]),
        _bd8(Wb_nbr[0, 0][:, HALF:]), jnp.tile(bb[0, 0], UP).reshape(1, G_OUT))
    ylo, yhi = _sc_wide(_flat16(tlo), _flat16(thi), src, dst, zer)

    for l in range(L_BLOCK):
        _, s, tlo, thi = _cproj(
            s, _grouped(ylo), _grouped(yhi), plo, phi,
            _bd8(Wb_self[l, 1]), _bd8(Wb_nbr[l, 1][:, :HALF]),
            _bd8(Wb_nbr[l, 1][:, HALF:]),
            jnp.tile(bb[l, 1], UP).reshape(1, G_OUT))
        ylo, yhi = _sc_wide(_flat16(tlo), _flat16(thi), src, dst, zer)
        if l < L_BLOCK - 1:
            hres, s, tlo, thi = _cproj_res(
                hres, s, _grouped(ylo), _grouped(yhi), plo, phi,
                _bd8(Wb_self[l + 1, 0]), _bd8(Wb_nbr[l + 1, 0][:, :HALF]),
                _bd8(Wb_nbr[l + 1, 0][:, HALF:]),
                jnp.tile(bb[l + 1, 0], UP).reshape(1, G_OUT))
            ylo, yhi = _sc_wide(_flat16(tlo), _flat16(thi), src, dst, zer)

    e0 = jnp.zeros((1, HALF), f32).at[0, 0].set(1.0)
    wcs = _bd8(Wc_self @ e0)
    wcn = _bd8(Wc_nbr @ e0)
    bc_g = jnp.tile(bc[0] * e0, (1, UP))
    hfin, sc_g, tpad = _ccls(hres, s, _grouped(ylo), _grouped(yhi), plo, phi,
                             wcs, wcn, bc_g)
    y0, y1 = _sc_cls(_flat16(tpad), src, dst, zer)

    lg_g, mx, top = _finreduce(sc_g, _grouped(y0), _grouped(y1))

    p8 = jnp.zeros((UP, G_HALF), f32).at[
        jnp.arange(UP), HALF * jnp.arange(UP)].set(1.0)
    e128 = jnp.zeros((G_HALF, G_OUT), f32).at[HALF * jnp.arange(UP)].set(
        jnp.kron(jnp.eye(UP, dtype=f32), jnp.ones((1, C_OUT), f32)))
    tgt8 = target_label.astype(f32).reshape(N_IN, UP)
    outg, keep_g = _mask(hfin, lg_g, tgt8, p8, e128, mx, top)
    out_pruned = outg.reshape(N_UP, C_OUT)
    out_cls = lg_g.reshape(N_UP, HALF)[:, :1]
    keep = keep_g.reshape(N_UP, HALF)[:, 0] != 0
    return out_pruned, out_cls, target_label, keep
